# baseline (speedup 1.0000x reference)
import functools

import jax
import jax.numpy as jnp
from jax import lax
from jax.experimental import pallas as pl
from jax.experimental.pallas import tpu as pltpu
from jax.experimental.pallas import tpu_sc as plsc

D_MODEL = 2048
SEQ_LEN = 2048
BATCH = 4

NC = 2
NS = 16
NW = NC * NS
LANES = 16

ROWS = BATCH * SEQ_LEN
ROWS_PER_W = ROWS // NW
CH = 16
NCHUNK = ROWS_PER_W // CH
GROUPS = D_MODEL // LANES


def _body(ids_hbm, tok_hbm, pos_hbm, out_hbm, idx_v, tok_buf, pos_buf, gsem, psem):
    wid = lax.axis_index("s") * NC + lax.axis_index("c")
    base = wid * ROWS_PER_W
    s0 = base % SEQ_LEN

    pltpu.sync_copy(ids_hbm.at[wid], idx_v)

    zeros_i = jnp.zeros((LANES,), jnp.int32)
    ones_f = jnp.ones((LANES,), jnp.float32)
    zeros_f = jnp.zeros((LANES,), jnp.float32)

    def chunk_step(c, carry):
        off = c * CH
        g = pltpu.async_copy(tok_hbm.at[idx_v.at[c]], tok_buf, gsem)
        p = pltpu.async_copy(pos_hbm.at[pl.ds(s0 + off, CH)], pos_buf, psem)
        g.wait()
        p.wait()

        idvec = idx_v[c]
        masks = []
        for r in range(CH):
            idx_splat = jnp.broadcast_to(idvec[r], (LANES,))
            masks.append(jnp.where(idx_splat != zeros_i, ones_f, zeros_f))

        def group_step(j, carry2):
            col = j * LANES
            for r in range(CH):
                t = tok_buf[r, pl.ds(col, LANES)]
                pv = pos_buf[r, pl.ds(col, LANES)]
                tok_buf[r, pl.ds(col, LANES)] = (t + pv) * masks[r]
            return carry2

        lax.fori_loop(0, GROUPS, group_step, 0)

        pltpu.sync_copy(tok_buf, out_hbm.at[pl.ds(base + off, CH)])
        return carry

    lax.fori_loop(0, NCHUNK, chunk_step, 0)


@jax.jit
def _run(ids_grouped, token_table, pos_table):
    mesh = plsc.VectorSubcoreMesh(core_axis_name="c", subcore_axis_name="s")
    f = functools.partial(
        pl.kernel,
        mesh=mesh,
        out_type=jax.ShapeDtypeStruct((ROWS, D_MODEL), jnp.float32),
        scratch_types=[
            pltpu.VMEM((NCHUNK, CH), jnp.int32),
            pltpu.VMEM((CH, D_MODEL), jnp.float32),
            pltpu.VMEM((CH, D_MODEL), jnp.float32),
            pltpu.SemaphoreType.DMA,
            pltpu.SemaphoreType.DMA,
        ],
    )(_body)
    return f(ids_grouped, token_table, pos_table)


def kernel(input_ids, token_table, pos_table):
    ids_grouped = input_ids.reshape(NW, NCHUNK, CH).astype(jnp.int32)
    out = _run(ids_grouped, token_table, pos_table)
    return out.reshape(BATCH, SEQ_LEN, D_MODEL)


# speedup vs baseline: 1.5576x; 1.5576x over previous
import functools

import jax
import jax.numpy as jnp
from jax import lax
from jax.experimental import pallas as pl
from jax.experimental.pallas import tpu as pltpu
from jax.experimental.pallas import tpu_sc as plsc

D_MODEL = 2048
SEQ_LEN = 2048
BATCH = 4

NC = 2
NS = 16
NW = NC * NS
LANES = 16

ROWS = BATCH * SEQ_LEN
ROWS_PER_W = ROWS // NW
CH = 8
NCHUNK = ROWS_PER_W // CH
OUTER = NCHUNK // 2
GROUPS = D_MODEL // LANES


def _body(ids2_hbm, ids1_hbm, tok_hbm, pos_hbm, out_hbm,
          idx2_v, idx1_v, tok0, tok1, pos0, pos1, ob0, ob1,
          gsem0, gsem1, psem0, psem1, ssem0, ssem1):
    wid = lax.axis_index("s") * NC + lax.axis_index("c")
    base = wid * ROWS_PER_W
    s0 = base % SEQ_LEN

    tok = (tok0, tok1)
    pos = (pos0, pos1)
    ob = (ob0, ob1)
    gsem = (gsem0, gsem1)
    psem = (psem0, psem1)
    ssem = (ssem0, ssem1)

    pltpu.sync_copy(ids2_hbm.at[wid], idx2_v)
    pltpu.sync_copy(ids1_hbm.at[pl.ds(base, ROWS_PER_W)],
                    idx1_v.at[pl.ds(0, ROWS_PER_W)])

    zeros_i = jnp.zeros((LANES,), jnp.int32)
    ones_f = jnp.ones((LANES,), jnp.float32)
    zeros_f = jnp.zeros((LANES,), jnp.float32)

    def fire(c, b):
        pltpu.async_copy(tok_hbm.at[idx2_v.at[c]], tok[b], gsem[b])
        pltpu.async_copy(pos_hbm.at[pl.ds(s0 + c * CH, CH)], pos[b], psem[b])

    fire(0, 0)
    fire(1, 1)

    @pl.loop(0, OUTER)
    def _(it):
        idvec = idx1_v[pl.ds(it * 2 * CH, LANES)]
        for b in range(2):
            c = it * 2 + b
            pltpu.make_async_copy(tok_hbm.at[idx2_v.at[c]], tok[b], gsem[b]).wait()
            pltpu.make_async_copy(
                pos_hbm.at[pl.ds(s0, CH)], pos[b], psem[b]
            ).wait()

            @pl.when(it > 0)
            def _():
                pltpu.make_async_copy(
                    ob[b], out_hbm.at[pl.ds(base, CH)], ssem[b]
                ).wait()

            masks = [
                jnp.where(
                    jnp.broadcast_to(idvec[b * CH + r], (LANES,)) != zeros_i,
                    ones_f, zeros_f)
                for r in range(CH)
            ]

            def group_step(j, carry2):
                col = j * LANES
                for r in range(CH):
                    t = tok[b][r, pl.ds(col, LANES)]
                    pv = pos[b][r, pl.ds(col, LANES)]
                    ob[b][r, pl.ds(col, LANES)] = (t + pv) * masks[r]
                return carry2

            lax.fori_loop(0, GROUPS, group_step, 0)

            pltpu.async_copy(ob[b], out_hbm.at[pl.ds(base + c * CH, CH)], ssem[b])

            @pl.when(it < OUTER - 1)
            def _():
                fire(c + 2, b)

    for b in range(2):
        pltpu.make_async_copy(ob[b], out_hbm.at[pl.ds(base, CH)], ssem[b]).wait()


@jax.jit
def _run(ids2, ids1, token_table, pos_table):
    mesh = plsc.VectorSubcoreMesh(core_axis_name="c", subcore_axis_name="s")
    f = functools.partial(
        pl.kernel,
        mesh=mesh,
        out_type=jax.ShapeDtypeStruct((ROWS, D_MODEL), jnp.float32),
        scratch_types=[
            pltpu.VMEM((NCHUNK, CH), jnp.int32),
            pltpu.VMEM((ROWS_PER_W + LANES,), jnp.int32),
            pltpu.VMEM((CH, D_MODEL), jnp.float32),
            pltpu.VMEM((CH, D_MODEL), jnp.float32),
            pltpu.VMEM((CH, D_MODEL), jnp.float32),
            pltpu.VMEM((CH, D_MODEL), jnp.float32),
            pltpu.VMEM((CH, D_MODEL), jnp.float32),
            pltpu.VMEM((CH, D_MODEL), jnp.float32),
            pltpu.SemaphoreType.DMA,
            pltpu.SemaphoreType.DMA,
            pltpu.SemaphoreType.DMA,
            pltpu.SemaphoreType.DMA,
            pltpu.SemaphoreType.DMA,
            pltpu.SemaphoreType.DMA,
        ],
    )(_body)
    return f(ids2, ids1, token_table, pos_table)


def kernel(input_ids, token_table, pos_table):
    ids1 = input_ids.reshape(-1).astype(jnp.int32)
    ids2 = ids1.reshape(NW, NCHUNK, CH)
    out = _run(ids2, ids1, token_table, pos_table)
    return out.reshape(BATCH, SEQ_LEN, D_MODEL)


# speedup vs baseline: 2.1482x; 1.3792x over previous
import functools

import jax
import jax.numpy as jnp
from jax import lax
from jax.experimental import pallas as pl
from jax.experimental.pallas import tpu as pltpu
from jax.experimental.pallas import tpu_sc as plsc

D_MODEL = 2048
SEQ_LEN = 2048
BATCH = 4

NC = 2
NS = 16
NW = NC * NS
LANES = 16

ROWS = BATCH * SEQ_LEN
ROWS_PER_W = ROWS // NW
S_PER_W = SEQ_LEN // NW
CH = 8
NCHUNK = S_PER_W // CH
NSTEP = NCHUNK * BATCH
OUTER = NCHUNK // 2
GROUPS = D_MODEL // LANES


def _body(ids2_hbm, ids1_hbm, tok_hbm, pos_hbm, out_hbm,
          idx2_v, idx1_v, tok0, tok1, pos0, pos1, ob0, ob1,
          gsem0, gsem1, psem0, psem1, ssem0, ssem1):
    wid = lax.axis_index("s") * NC + lax.axis_index("c")
    base = wid * ROWS_PER_W
    wof = wid * S_PER_W

    tok = (tok0, tok1)
    pos = (pos0, pos1)
    ob = (ob0, ob1)
    gsem = (gsem0, gsem1)
    psem = (psem0, psem1)
    ssem = (ssem0, ssem1)

    pltpu.sync_copy(ids2_hbm.at[wid], idx2_v)
    pltpu.sync_copy(ids1_hbm.at[pl.ds(base, ROWS_PER_W)],
                    idx1_v.at[pl.ds(0, ROWS_PER_W)])

    zeros_i = jnp.zeros((LANES,), jnp.int32)
    ones_f = jnp.ones((LANES,), jnp.float32)
    zeros_f = jnp.zeros((LANES,), jnp.float32)

    def fire_gather(step, buf):
        pltpu.async_copy(tok_hbm.at[idx2_v.at[step]], tok[buf], gsem[buf])

    def fire_pos(c, buf):
        pltpu.async_copy(pos_hbm.at[pl.ds(wof + c * CH, CH)], pos[buf], psem[buf])

    fire_gather(0, 0)
    fire_gather(1, 1)
    fire_pos(0, 0)
    fire_pos(1, 1)

    @pl.loop(0, OUTER)
    def _(cc):
        for cpar in range(2):
            c = cc * 2 + cpar
            coff = c * (BATCH * CH)
            idlo = idx1_v[pl.ds(coff, LANES)]
            idhi = idx1_v[pl.ds(coff + LANES, LANES)]

            pltpu.make_async_copy(
                pos_hbm.at[pl.ds(wof, CH)], pos[cpar], psem[cpar]
            ).wait()

            for b in range(BATCH):
                buf = b % 2
                step = c * BATCH + b

                pltpu.make_async_copy(
                    tok_hbm.at[idx2_v.at[step]], tok[buf], gsem[buf]
                ).wait()

                def drain():
                    pltpu.make_async_copy(
                        ob[buf], out_hbm.at[pl.ds(base, CH)], ssem[buf]
                    ).wait()

                if cpar == 0 and b < 2:
                    pl.when(cc > 0)(drain)
                else:
                    drain()

                idvec = idlo if b < 2 else idhi
                masks = [
                    jnp.where(
                        jnp.broadcast_to(idvec[(b % 2) * CH + r], (LANES,))
                        != zeros_i,
                        ones_f, zeros_f)
                    for r in range(CH)
                ]

                def group_step(j, carry2):
                    col = j * LANES
                    for r in range(CH):
                        t = tok[buf][r, pl.ds(col, LANES)]
                        pv = pos[cpar][r, pl.ds(col, LANES)]
                        ob[buf][r, pl.ds(col, LANES)] = (t + pv) * masks[r]
                    return carry2

                lax.fori_loop(0, GROUPS, group_step, 0)

                pltpu.async_copy(
                    ob[buf],
                    out_hbm.at[pl.ds(b * SEQ_LEN + wof + c * CH, CH)],
                    ssem[buf],
                )

                if b < 2:
                    fire_gather(step + 2, buf)
                elif cpar == 0:
                    fire_gather(step + 2, buf)
                else:
                    pl.when(cc < OUTER - 1)(lambda: fire_gather(step + 2, buf))

            pl.when(cc < OUTER - 1)(lambda: fire_pos(c + 2, cpar))

    for buf in range(2):
        pltpu.make_async_copy(
            ob[buf], out_hbm.at[pl.ds(base, CH)], ssem[buf]
        ).wait()


@jax.jit
def _run(ids2, ids1, token_table, pos_table):
    mesh = plsc.VectorSubcoreMesh(core_axis_name="c", subcore_axis_name="s")
    f = functools.partial(
        pl.kernel,
        mesh=mesh,
        out_type=jax.ShapeDtypeStruct((ROWS, D_MODEL), jnp.float32),
        scratch_types=[
            pltpu.VMEM((NSTEP, CH), jnp.int32),
            pltpu.VMEM((ROWS_PER_W + LANES,), jnp.int32),
            pltpu.VMEM((CH, D_MODEL), jnp.float32),
            pltpu.VMEM((CH, D_MODEL), jnp.float32),
            pltpu.VMEM((CH, D_MODEL), jnp.float32),
            pltpu.VMEM((CH, D_MODEL), jnp.float32),
            pltpu.VMEM((CH, D_MODEL), jnp.float32),
            pltpu.VMEM((CH, D_MODEL), jnp.float32),
            pltpu.SemaphoreType.DMA,
            pltpu.SemaphoreType.DMA,
            pltpu.SemaphoreType.DMA,
            pltpu.SemaphoreType.DMA,
            pltpu.SemaphoreType.DMA,
            pltpu.SemaphoreType.DMA,
        ],
    )(_body)
    return f(ids2, ids1, token_table, pos_table)


def kernel(input_ids, token_table, pos_table):
    ids_r = (
        input_ids.astype(jnp.int32)
        .reshape(BATCH, NW, NCHUNK, CH)
        .transpose(1, 2, 0, 3)
    )
    ids1 = ids_r.reshape(-1)
    ids2 = ids_r.reshape(NW, NSTEP, CH)
    out = _run(ids2, ids1, token_table, pos_table)
    return out.reshape(BATCH, SEQ_LEN, D_MODEL)
